# baseline (speedup 1.0000x reference)
import functools

import jax
import jax.numpy as jnp
from jax import lax
from jax.experimental import pallas as pl
from jax.experimental.pallas import tpu as pltpu
from jax.experimental.pallas import tpu_sc as plsc

N = 10000
E = 320000
IN_SIZE = 128
OUT_SIZE = 20
SIZE = 128
SEQ_SIZE = 32
ATT = 16
HEADS = 4
HA = HEADS * ATT
KERNELS = 16
MAX_DIST = 20.0

NC = 2
NS = 16
NW = NC * NS
CHUNK = 128
MC = 4
NCH = E // CHUNK
RPW = 80
NCHP = NW * RPW
MCS = 2
NTILE = N // NS
ACCW = 80

EB = 2560
NBK = 512


def _sc_gather2(tab1, idx1p, tab2, idx2p):
    d1 = tab1.shape[1]
    d2 = tab2.shape[1]
    mesh = plsc.VectorSubcoreMesh(core_axis_name="c", subcore_axis_name="s", num_cores=NC, num_subcores=NS)
    out_type = (
        jax.ShapeDtypeStruct((NCH, CHUNK, d1), tab1.dtype),
        jax.ShapeDtypeStruct((NCH, CHUNK, d2), tab2.dtype),
    )
    scratch = [
        pltpu.VMEM((RPW, CHUNK), jnp.int32),
        pltpu.VMEM((RPW, CHUNK), jnp.int32),
        pltpu.VMEM((MC, CHUNK, d1), tab1.dtype),
        pltpu.VMEM((MC, CHUNK, d2), tab2.dtype),
        pltpu.SemaphoreType.DMA,
        pltpu.SemaphoreType.DMA,
    ]

    def body(t1_h, i1_h, t2_h, i2_h, o1_h, o2_h, i1_v, i2_v, b1_v, b2_v, s1, s2):
        c = lax.axis_index("c")
        s = lax.axis_index("s")
        wid = s * NC + c
        row0 = wid * RPW
        pltpu.sync_copy(i1_h.at[pl.ds(row0, RPW)], i1_v)
        pltpu.sync_copy(i2_h.at[pl.ds(row0, RPW)], i2_v)
        nj = jnp.minimum(RPW // MC, jnp.maximum(0, (NCH - row0) // MC))

        def jbody(j, carry):
            base = j * MC
            gbase = row0 + base
            waits = []
            for i in range(MC):
                waits.append(pltpu.async_copy(t1_h.at[i1_v.at[base + i]], b1_v.at[i], s1))
                waits.append(pltpu.async_copy(t2_h.at[i2_v.at[base + i]], b2_v.at[i], s2))
            for w in waits:
                w.wait()
            pltpu.sync_copy(b1_v, o1_h.at[pl.ds(gbase, MC)])
            pltpu.sync_copy(b2_v, o2_h.at[pl.ds(gbase, MC)])
            return carry

        lax.fori_loop(0, nj, jbody, 0)

    f = pl.kernel(body, out_type=out_type, mesh=mesh, scratch_types=scratch,
                  compiler_params=pltpu.CompilerParams(use_tc_tiling_on_sc=False))
    return f(tab1, idx1p, tab2, idx2p)


def _sc_scatter(wv3, ind2, zeros_h):
    mesh = plsc.VectorSubcoreMesh(core_axis_name="c", subcore_axis_name="s", num_cores=NC, num_subcores=NS)
    out_type = jax.ShapeDtypeStruct((NC, N, ACCW), jnp.float32)
    scratch = [
        pltpu.VMEM_SHARED((N, ACCW), jnp.float32),
        pltpu.VMEM((RPW, CHUNK), jnp.int32),
        pltpu.VMEM((2, MCS, CHUNK, ACCW), jnp.float32),
        pltpu.SemaphoreType.DMA((2,)),
    ]

    def body(wv_h, ind_h, z_h, part_h, acc, ind_v, wv_v, sem):
        c = lax.axis_index("c")
        s = lax.axis_index("s")
        wid = s * NC + c
        r0 = s * NTILE
        row0 = wid * RPW
        pltpu.sync_copy(z_h.at[pl.ds(r0, NTILE)], acc.at[pl.ds(r0, NTILE)])
        pltpu.sync_copy(ind_h.at[pl.ds(row0, RPW)], ind_v)
        plsc.subcore_barrier()
        nj = jnp.minimum(RPW // MCS, jnp.maximum(0, (NCH - row0) // MCS))

        def adds(base, slot):
            for i in range(MCS):
                pltpu.sync_copy(wv_v.at[slot, i], acc.at[ind_v.at[base + i]], add=True)

        def jbody(jj, carry):
            j0 = 2 * jj
            ha = pltpu.async_copy(wv_h.at[pl.ds(row0 + j0 * MCS, MCS)], wv_v.at[0], sem.at[0])

            @pl.when(j0 + 1 < nj)
            def _():
                pltpu.async_copy(wv_h.at[pl.ds(row0 + (j0 + 1) * MCS, MCS)], wv_v.at[1], sem.at[1])

            ha.wait()
            adds(j0 * MCS, 0)

            @pl.when(j0 + 1 < nj)
            def _():
                pltpu.make_async_copy(wv_h.at[pl.ds(0, MCS)], wv_v.at[1], sem.at[1]).wait()
                adds((j0 + 1) * MCS, 1)

            return carry

        lax.fori_loop(0, (nj + 1) // 2, jbody, 0)
        plsc.subcore_barrier()
        pltpu.sync_copy(acc.at[pl.ds(r0, NTILE)], part_h.at[c, pl.ds(r0, NTILE)])

    f = pl.kernel(body, out_type=out_type, mesh=mesh, scratch_types=scratch,
                  compiler_params=pltpu.CompilerParams(use_tc_tiling_on_sc=False))
    return f(wv3, ind2, zeros_h)


def _full(shape):
    return pl.BlockSpec(shape, lambda i: tuple(0 for _ in shape))


def _rows(bs, width):
    return pl.BlockSpec((bs, width), lambda i: (i, 0))


def _rel_tc(src_d, tgt_d):
    def body(src_r, tgt_r, out_r):
        src = src_r[...]
        tgt = tgt_r[...]
        diff = tgt[:, 0:3] - src[:, 0:3]
        d2 = jnp.sum(diff * diff, axis=1, keepdims=True)
        dist = jnp.sqrt(d2 + 1e-12)
        direction = diff / (dist + 1e-6)
        centers = jnp.arange(KERNELS, dtype=jnp.int32).astype(jnp.float32) * (MAX_DIST / (KERNELS - 1))
        sigma = MAX_DIST / KERNELS
        rbf = jnp.exp(-((dist - centers[None, :]) ** 2) / (2.0 * sigma * sigma))
        rots = []
        for i in range(3):
            xo = tgt[:, 3 + 3 * i:6 + 3 * i]
            for j in range(3):
                yo = src[:, 3 + 3 * j:6 + 3 * j]
                rots.append(jnp.sum(xo * yo, axis=1, keepdims=True))
        di = (tgt[:, 12:13] - src[:, 12:13]) * 0.1
        ds = jnp.sin(di)
        dc = jnp.cos(di)
        pad = jnp.zeros((src.shape[0], 2), jnp.float32)
        out_r[...] = jnp.concatenate([rbf, direction] + rots + [ds, dc, pad],
                                     axis=1).astype(jnp.bfloat16)

    return pl.pallas_call(
        body,
        grid=(E // EB,),
        in_specs=[_rows(EB, 16), _rows(EB, 16)],
        out_specs=_rows(EB, 32),
        out_shape=jax.ShapeDtypeStruct((E, 32), jnp.bfloat16),
    )(src_d, tgt_d)


def _edge_tc(kv_g, q_g, rel, wkr, wvr):
    def body(kv_r, q_r, rel_r, wkr_r, wvr_r, out_r):
        kv = kv_r[...].astype(jnp.float32)
        q = q_r[...].astype(jnp.float32)
        relv = rel_r[...].astype(jnp.float32)
        k = kv[:, :HA] + jnp.dot(relv, wkr_r[...], preferred_element_type=jnp.float32)
        v = kv[:, HA:] + jnp.dot(relv, wvr_r[...], preferred_element_type=jnp.float32)
        t = q * k
        dd = lax.broadcasted_iota(jnp.int32, (HA, HEADS), 0) // ATT
        hh = lax.broadcasted_iota(jnp.int32, (HA, HEADS), 1)
        collapse = (dd == hh).astype(jnp.float32)
        logits = jnp.dot(t, collapse, preferred_element_type=jnp.float32) * 0.25
        w = jnp.exp(logits)
        wb = jnp.dot(w, collapse.T, preferred_element_type=jnp.float32)
        wv = wb * v
        pad = jnp.zeros((kv.shape[0], ACCW - HA - HEADS), jnp.float32)
        out_r[...] = jnp.concatenate([wv, w, pad], axis=1)

    return pl.pallas_call(
        body,
        grid=(E // EB,),
        in_specs=[_rows(EB, 128), _rows(EB, HA), _rows(EB, 32),
                  _full((32, HA)), _full((32, HA))],
        out_specs=_rows(EB, ACCW),
        out_shape=jax.ShapeDtypeStruct((E, ACCW), jnp.float32),
    )(kv_g, q_g, rel, wkr, wvr)


def _pre_tc(features, onehot, wpre, bpre, wse, bse):
    def body(f_r, oh_r, wp_r, bp_r, ws_r, bs_r, out_r, seq_r):
        out_r[...] = jnp.dot(f_r[...], wp_r[...],
                             preferred_element_type=jnp.float32) + bp_r[...]
        seq_r[...] = jnp.dot(oh_r[...], ws_r[...],
                             preferred_element_type=jnp.float32) + bs_r[...]

    grid = (pl.cdiv(N, NBK),)
    return pl.pallas_call(
        body,
        grid=grid,
        in_specs=[_rows(NBK, IN_SIZE), _rows(NBK, OUT_SIZE),
                  _full((IN_SIZE, SIZE)), _full((1, SIZE)),
                  _full((OUT_SIZE, SEQ_SIZE)), _full((1, SEQ_SIZE))],
        out_specs=[_rows(NBK, SIZE), _rows(NBK, SEQ_SIZE)],
        out_shape=[jax.ShapeDtypeStruct((N, SIZE), jnp.float32),
                   jax.ShapeDtypeStruct((N, SEQ_SIZE), jnp.float32)],
    )(features, onehot, wpre, bpre, wse, bse)


def _node_prep_enc(feat, w0, b0, w1, b1, wkv, bkv, wq, bq):
    def body(f_r, w0r, b0r, w1r, b1r, wkvr, bkvr, wqr, bqr, inp_r, kv_r, qn_r):
        inp = jnp.maximum(f_r[...], 0.0)
        h = jnp.maximum(jnp.dot(inp, w0r[...], preferred_element_type=jnp.float32) + b0r[...], 0.0)
        local = jnp.maximum(jnp.dot(h, w1r[...], preferred_element_type=jnp.float32) + b1r[...], 0.0)
        inp_r[...] = inp
        kv_r[...] = (jnp.dot(local, wkvr[...], preferred_element_type=jnp.float32)
                     + bkvr[...]).astype(jnp.bfloat16)
        qn_r[...] = (jnp.dot(local, wqr[...], preferred_element_type=jnp.float32)
                     + bqr[...]).astype(jnp.bfloat16)

    grid = (pl.cdiv(N, NBK),)
    return pl.pallas_call(
        body,
        grid=grid,
        in_specs=[_rows(NBK, SIZE),
                  _full((SIZE, SIZE)), _full((1, SIZE)),
                  _full((SIZE, SIZE)), _full((1, SIZE)),
                  _full((SIZE, 2 * HA)), _full((1, 2 * HA)),
                  _full((SIZE, HA)), _full((1, HA))],
        out_specs=[_rows(NBK, SIZE), _rows(NBK, 2 * HA), _rows(NBK, HA)],
        out_shape=[jax.ShapeDtypeStruct((N, SIZE), jnp.float32),
                   jax.ShapeDtypeStruct((N, 2 * HA), jnp.bfloat16),
                   jax.ShapeDtypeStruct((N, HA), jnp.bfloat16)],
    )(feat, w0, b0, w1, b1, wkv, bkv, wq, bq)


def _node_prep_dec(feat, enc, seqtab, w0, b0, w1, b1, wkv, bkv, wkvs, wq, bq):
    def body(f_r, e_r, sq_r, w0r, b0r, w1r, b1r, wkvr, bkvr, wkvsr, wqr, bqr,
             inp_r, kva_r, kvb_r, qn_r):
        inp = jnp.maximum(f_r[...], 0.0)
        h = jnp.maximum(jnp.dot(inp, w0r[...], preferred_element_type=jnp.float32) + b0r[...], 0.0)
        local = jnp.maximum(jnp.dot(h, w1r[...], preferred_element_type=jnp.float32) + b1r[...], 0.0)
        inp_r[...] = inp
        kva_r[...] = (jnp.dot(local, wkvr[...], preferred_element_type=jnp.float32)
                      + jnp.dot(sq_r[...], wkvsr[...], preferred_element_type=jnp.float32)
                      + bkvr[...]).astype(jnp.bfloat16)
        kvb_r[...] = (jnp.dot(e_r[...], wkvr[...], preferred_element_type=jnp.float32)
                      + bkvr[...]).astype(jnp.bfloat16)
        qn_r[...] = (jnp.dot(local, wqr[...], preferred_element_type=jnp.float32)
                     + bqr[...]).astype(jnp.bfloat16)

    grid = (pl.cdiv(N, NBK),)
    return pl.pallas_call(
        body,
        grid=grid,
        in_specs=[_rows(NBK, SIZE), _rows(NBK, SIZE), _rows(NBK, SEQ_SIZE),
                  _full((SIZE, SIZE)), _full((1, SIZE)),
                  _full((SIZE, SIZE)), _full((1, SIZE)),
                  _full((SIZE, 2 * HA)), _full((1, 2 * HA)),
                  _full((SEQ_SIZE, 2 * HA)),
                  _full((SIZE, HA)), _full((1, HA))],
        out_specs=[_rows(NBK, SIZE), _rows(NBK, 2 * HA), _rows(NBK, 2 * HA),
                   _rows(NBK, HA)],
        out_shape=[jax.ShapeDtypeStruct((N, SIZE), jnp.float32),
                   jax.ShapeDtypeStruct((N, 2 * HA), jnp.bfloat16),
                   jax.ShapeDtypeStruct((N, 2 * HA), jnp.bfloat16),
                   jax.ShapeDtypeStruct((N, HA), jnp.bfloat16)],
    )(feat, enc, seqtab, w0, b0, w1, b1, wkv, bkv, wkvs, wq, bq)


def _node_post(part, inp, wo, bo, wpost=None, bpost=None):
    final = wpost is not None

    def body(*args):
        if final:
            p_r, inp_r, wo_r, bo_r, wp_r, bp_r, out_r = args
        else:
            p_r, inp_r, wo_r, bo_r, out_r = args
        p = p_r[...]
        s = p[0] + p[1]
        ii = lax.broadcasted_iota(jnp.int32, (ACCW, HA), 0)
        jj = lax.broadcasted_iota(jnp.int32, (ACCW, HA), 1)
        pick_num = jnp.where((ii == jj) & (ii < HA), 1.0, 0.0)
        pick_den = jnp.where((ii >= HA) & (ii < HA + HEADS) & ((ii - HA) == jj // ATT), 1.0, 0.0)
        num = jnp.dot(s, pick_num, preferred_element_type=jnp.float32)
        den = jnp.dot(s, pick_den, preferred_element_type=jnp.float32)
        att_in = num / (den + 1e-30)
        att = jnp.dot(att_in, wo_r[...], preferred_element_type=jnp.float32) + bo_r[...]
        feat = inp_r[...] + att
        if final:
            out_r[...] = jnp.dot(feat, wp_r[...], preferred_element_type=jnp.float32) + bp_r[...]
        else:
            out_r[...] = feat

    grid = (pl.cdiv(N, NBK),)
    in_specs = [pl.BlockSpec((NC, NBK, ACCW), lambda i: (0, i, 0)),
                _rows(NBK, SIZE), _full((HA, SIZE)), _full((1, SIZE))]
    args = [part, inp, wo, bo]
    if final:
        in_specs += [_full((SIZE, OUT_SIZE)), _full((1, OUT_SIZE))]
        args += [wpost, bpost]
        out_w = OUT_SIZE
    else:
        out_w = SIZE
    return pl.pallas_call(
        body,
        grid=grid,
        in_specs=in_specs,
        out_specs=_rows(NBK, out_w),
        out_shape=jax.ShapeDtypeStruct((N, out_w), jnp.float32),
    )(*args)


def _row(b):
    return b.reshape(1, -1)


def kernel(features, distances, params, sequence, connections, indices):
    conn = connections.astype(jnp.int32)
    ind = indices.astype(jnp.int32)
    pad_rows = ((0, NCHP - NCH), (0, 0))
    conn2 = jnp.pad(conn.reshape(NCH, CHUNK), pad_rows)
    ind2 = jnp.pad(ind.reshape(NCH, CHUNK), pad_rows)

    distp = jnp.pad(distances, ((0, 0), (0, 3)))
    src3, tgt3 = _sc_gather2(distp, conn2, distp, ind2)
    rel = _rel_tc(src3.reshape(E, 16), tgt3.reshape(E, 16))

    onehot = jax.nn.one_hot(sequence, OUT_SIZE, dtype=jnp.float32)
    feat, seqtab = _pre_tc(features, onehot,
                           params["pre"]["W"], _row(params["pre"]["b"]),
                           params["seq_emb"]["W"], _row(params["seq_emb"]["b"]))

    zeros_acc = jnp.zeros((N, ACCW), jnp.float32)

    def block_weights(bp, msg_extra):
        wk, wv = bp["k"]["W"], bp["v"]["W"]
        wkv = jnp.concatenate([wk[:SIZE], wv[:SIZE]], axis=1)
        bkv = jnp.concatenate([bp["k"]["b"], bp["v"]["b"]])
        wkr = jnp.pad(wk[SIZE:SIZE + 30], ((0, 2), (0, 0)))
        wvr = jnp.pad(wv[SIZE:SIZE + 30], ((0, 2), (0, 0)))
        extra = None
        if msg_extra:
            extra = jnp.concatenate([wk[SIZE + 30:], wv[SIZE + 30:]], axis=1)
        return wkv, bkv, wkr, wvr, extra

    for bp in params["enc_blocks"]:
        wkv, bkv, wkr, wvr, _ = block_weights(bp, False)
        inp, kvt, qn = _node_prep_enc(
            feat, bp["local"][0]["W"], _row(bp["local"][0]["b"]),
            bp["local"][1]["W"], _row(bp["local"][1]["b"]),
            wkv, _row(bkv), bp["q"]["W"], _row(bp["q"]["b"]))
        kv3, q3 = _sc_gather2(kvt, conn2, qn, ind2)
        wvx = _edge_tc(kv3.reshape(E, 2 * HA), q3.reshape(E, HA), rel, wkr, wvr)
        part = _sc_scatter(wvx.reshape(NCH, CHUNK, ACCW), ind2, zeros_acc)
        feat = _node_post(part, inp, bp["o"]["W"], _row(bp["o"]["b"]))

    encoding = feat

    pre_mask = conn < jnp.arange(E, dtype=jnp.int32)
    idx_dec2 = jnp.pad(jnp.where(pre_mask, conn, conn + N).reshape(NCH, CHUNK), pad_rows)

    for bi, bp in enumerate(params["dec_blocks"]):
        wkv, bkv, wkr, wvr, wkvs = block_weights(bp, True)
        inp, kva, kvb, qn = _node_prep_dec(
            feat, encoding, seqtab,
            bp["local"][0]["W"], _row(bp["local"][0]["b"]),
            bp["local"][1]["W"], _row(bp["local"][1]["b"]),
            wkv, _row(bkv), wkvs, bp["q"]["W"], _row(bp["q"]["b"]))
        kvt = jnp.concatenate([kva, kvb], axis=0)
        kv3, q3 = _sc_gather2(kvt, idx_dec2, qn, ind2)
        wvx = _edge_tc(kv3.reshape(E, 2 * HA), q3.reshape(E, HA), rel, wkr, wvr)
        part = _sc_scatter(wvx.reshape(NCH, CHUNK, ACCW), ind2, zeros_acc)
        if bi == len(params["dec_blocks"]) - 1:
            feat = _node_post(part, inp, bp["o"]["W"], _row(bp["o"]["b"]),
                              params["post"]["W"], _row(params["post"]["b"]))
        else:
            feat = _node_post(part, inp, bp["o"]["W"], _row(bp["o"]["b"]))

    return feat


# speedup vs baseline: 1.2792x; 1.2792x over previous
import functools

import jax
import jax.numpy as jnp
from jax import lax
from jax.experimental import pallas as pl
from jax.experimental.pallas import tpu as pltpu
from jax.experimental.pallas import tpu_sc as plsc

N = 10000
E = 320000
IN_SIZE = 128
OUT_SIZE = 20
SIZE = 128
SEQ_SIZE = 32
ATT = 16
HEADS = 4
HA = HEADS * ATT
KERNELS = 16
MAX_DIST = 20.0

NC = 2
NS = 16
NW = NC * NS
CHUNK = 128
MC = 2
NCH = E // CHUNK
RPW = 80
NCHP = NW * RPW
MCS = 1
NTILE = N // NS
ACCW = 128

EB = 2560
NBK = 512


def _sc_gather2(tab1, idx1p, tab2, idx2p):
    d1 = tab1.shape[1]
    d2 = tab2.shape[1]
    mesh = plsc.VectorSubcoreMesh(core_axis_name="c", subcore_axis_name="s", num_cores=NC, num_subcores=NS)
    out_type = (
        jax.ShapeDtypeStruct((NCH, CHUNK, d1), tab1.dtype),
        jax.ShapeDtypeStruct((NCH, CHUNK, d2), tab2.dtype),
    )
    scratch = [
        pltpu.VMEM((RPW, CHUNK), jnp.int32),
        pltpu.VMEM((RPW, CHUNK), jnp.int32),
        pltpu.VMEM((MC, CHUNK, d1), tab1.dtype),
        pltpu.VMEM((MC, CHUNK, d2), tab2.dtype),
        pltpu.SemaphoreType.DMA,
        pltpu.SemaphoreType.DMA,
    ]

    def body(t1_h, i1_h, t2_h, i2_h, o1_h, o2_h, i1_v, i2_v, b1_v, b2_v, s1, s2):
        c = lax.axis_index("c")
        s = lax.axis_index("s")
        wid = s * NC + c
        row0 = wid * RPW
        pltpu.sync_copy(i1_h.at[pl.ds(row0, RPW)], i1_v)
        pltpu.sync_copy(i2_h.at[pl.ds(row0, RPW)], i2_v)
        nj = jnp.minimum(RPW // MC, jnp.maximum(0, (NCH - row0) // MC))

        def jbody(j, carry):
            base = j * MC
            gbase = row0 + base
            waits = []
            for i in range(MC):
                waits.append(pltpu.async_copy(t1_h.at[i1_v.at[base + i]], b1_v.at[i], s1))
                waits.append(pltpu.async_copy(t2_h.at[i2_v.at[base + i]], b2_v.at[i], s2))
            for w in waits:
                w.wait()
            pltpu.sync_copy(b1_v, o1_h.at[pl.ds(gbase, MC)])
            pltpu.sync_copy(b2_v, o2_h.at[pl.ds(gbase, MC)])
            return carry

        lax.fori_loop(0, nj, jbody, 0)

    f = pl.kernel(body, out_type=out_type, mesh=mesh, scratch_types=scratch,
                  compiler_params=pltpu.CompilerParams(use_tc_tiling_on_sc=False))
    return f(tab1, idx1p, tab2, idx2p)


def _sc_scatter(wv3, ind2, zeros_h):
    mesh = plsc.VectorSubcoreMesh(core_axis_name="c", subcore_axis_name="s", num_cores=NC, num_subcores=NS)
    out_type = jax.ShapeDtypeStruct((NC, N, ACCW), jnp.float32)
    scratch = [
        pltpu.VMEM_SHARED((N, ACCW), jnp.float32),
        pltpu.VMEM((RPW, CHUNK), jnp.int32),
        pltpu.VMEM((2, MCS, CHUNK, ACCW), jnp.float32),
        pltpu.SemaphoreType.DMA((2,)),
    ]

    def body(wv_h, ind_h, z_h, part_h, acc, ind_v, wv_v, sem):
        c = lax.axis_index("c")
        s = lax.axis_index("s")
        wid = s * NC + c
        r0 = s * NTILE
        row0 = wid * RPW
        pltpu.sync_copy(z_h.at[pl.ds(r0, NTILE)], acc.at[pl.ds(r0, NTILE)])
        pltpu.sync_copy(ind_h.at[pl.ds(row0, RPW)], ind_v)
        plsc.subcore_barrier()
        nj = jnp.minimum(RPW // MCS, jnp.maximum(0, (NCH - row0) // MCS))

        def adds(base, slot):
            for i in range(MCS):
                pltpu.sync_copy(wv_v.at[slot, i], acc.at[ind_v.at[base + i]], add=True)

        def jbody(jj, carry):
            j0 = 2 * jj
            ha = pltpu.async_copy(wv_h.at[pl.ds(row0 + j0 * MCS, MCS)], wv_v.at[0], sem.at[0])

            @pl.when(j0 + 1 < nj)
            def _():
                pltpu.async_copy(wv_h.at[pl.ds(row0 + (j0 + 1) * MCS, MCS)], wv_v.at[1], sem.at[1])

            ha.wait()
            adds(j0 * MCS, 0)

            @pl.when(j0 + 1 < nj)
            def _():
                pltpu.make_async_copy(wv_h.at[pl.ds(0, MCS)], wv_v.at[1], sem.at[1]).wait()
                adds((j0 + 1) * MCS, 1)

            return carry

        lax.fori_loop(0, (nj + 1) // 2, jbody, 0)
        plsc.subcore_barrier()
        pltpu.sync_copy(acc.at[pl.ds(r0, NTILE)], part_h.at[c, pl.ds(r0, NTILE)])

    f = pl.kernel(body, out_type=out_type, mesh=mesh, scratch_types=scratch,
                  compiler_params=pltpu.CompilerParams(use_tc_tiling_on_sc=False))
    return f(wv3, ind2, zeros_h)


def _full(shape):
    return pl.BlockSpec(shape, lambda i: tuple(0 for _ in shape))


def _rows(bs, width):
    return pl.BlockSpec((bs, width), lambda i: (i, 0))


def _rel_tc(src_d, tgt_d):
    def body(src_r, tgt_r, out_r):
        src = src_r[...]
        tgt = tgt_r[...]
        diff = tgt[:, 0:3] - src[:, 0:3]
        d2 = jnp.sum(diff * diff, axis=1, keepdims=True)
        dist = jnp.sqrt(d2 + 1e-12)
        direction = diff / (dist + 1e-6)
        centers = jnp.arange(KERNELS, dtype=jnp.int32).astype(jnp.float32) * (MAX_DIST / (KERNELS - 1))
        sigma = MAX_DIST / KERNELS
        rbf = jnp.exp(-((dist - centers[None, :]) ** 2) / (2.0 * sigma * sigma))
        rots = []
        for i in range(3):
            xo = tgt[:, 3 + 3 * i:6 + 3 * i]
            for j in range(3):
                yo = src[:, 3 + 3 * j:6 + 3 * j]
                rots.append(jnp.sum(xo * yo, axis=1, keepdims=True))
        di = (tgt[:, 12:13] - src[:, 12:13]) * 0.1
        ds = jnp.sin(di)
        dc = jnp.cos(di)
        pad = jnp.zeros((src.shape[0], 2), jnp.float32)
        out_r[...] = jnp.concatenate([rbf, direction] + rots + [ds, dc, pad], axis=1)

    return pl.pallas_call(
        body,
        grid=(E // EB,),
        in_specs=[_rows(EB, 128), _rows(EB, 128)],
        out_specs=_rows(EB, 32),
        out_shape=jax.ShapeDtypeStruct((E, 32), jnp.float32),
    )(src_d, tgt_d)


def _edge_tc(kv_g, q_g, rel, wkr, wvr):
    def body(kv_r, q_r, rel_r, wkr_r, wvr_r, out_r):
        kv = kv_r[...]
        q = q_r[...][:, :HA]
        relv = rel_r[...]
        k = kv[:, :HA] + jnp.dot(relv, wkr_r[...], preferred_element_type=jnp.float32)
        v = kv[:, HA:] + jnp.dot(relv, wvr_r[...], preferred_element_type=jnp.float32)
        t = q * k
        dd = lax.broadcasted_iota(jnp.int32, (HA, HEADS), 0) // ATT
        hh = lax.broadcasted_iota(jnp.int32, (HA, HEADS), 1)
        collapse = (dd == hh).astype(jnp.float32)
        logits = jnp.dot(t, collapse, preferred_element_type=jnp.float32) * 0.25
        w = jnp.exp(logits)
        wb = jnp.dot(w, collapse.T, preferred_element_type=jnp.float32)
        wv = wb * v
        pad = jnp.zeros((kv.shape[0], ACCW - HA - HEADS), jnp.float32)
        out_r[...] = jnp.concatenate([wv, w, pad], axis=1)

    return pl.pallas_call(
        body,
        grid=(E // EB,),
        in_specs=[_rows(EB, 128), _rows(EB, 128), _rows(EB, 32),
                  _full((32, HA)), _full((32, HA))],
        out_specs=_rows(EB, ACCW),
        out_shape=jax.ShapeDtypeStruct((E, ACCW), jnp.float32),
    )(kv_g, q_g, rel, wkr, wvr)


def _pre_tc(features, onehot, wpre, bpre, wse, bse):
    def body(f_r, oh_r, wp_r, bp_r, ws_r, bs_r, out_r, seq_r):
        out_r[...] = jnp.dot(f_r[...], wp_r[...],
                             preferred_element_type=jnp.float32) + bp_r[...]
        seq_r[...] = jnp.dot(oh_r[...], ws_r[...],
                             preferred_element_type=jnp.float32) + bs_r[...]

    grid = (pl.cdiv(N, NBK),)
    return pl.pallas_call(
        body,
        grid=grid,
        in_specs=[_rows(NBK, IN_SIZE), _rows(NBK, OUT_SIZE),
                  _full((IN_SIZE, SIZE)), _full((1, SIZE)),
                  _full((OUT_SIZE, SEQ_SIZE)), _full((1, SEQ_SIZE))],
        out_specs=[_rows(NBK, SIZE), _rows(NBK, SEQ_SIZE)],
        out_shape=[jax.ShapeDtypeStruct((N, SIZE), jnp.float32),
                   jax.ShapeDtypeStruct((N, SEQ_SIZE), jnp.float32)],
    )(features, onehot, wpre, bpre, wse, bse)


def _node_prep_enc(feat, w0, b0, w1, b1, wkv, bkv, wq, bq):
    def body(f_r, w0r, b0r, w1r, b1r, wkvr, bkvr, wqr, bqr, inp_r, kv_r, qn_r):
        inp = jnp.maximum(f_r[...], 0.0)
        h = jnp.maximum(jnp.dot(inp, w0r[...], preferred_element_type=jnp.float32) + b0r[...], 0.0)
        local = jnp.maximum(jnp.dot(h, w1r[...], preferred_element_type=jnp.float32) + b1r[...], 0.0)
        inp_r[...] = inp
        kv_r[...] = jnp.dot(local, wkvr[...], preferred_element_type=jnp.float32) + bkvr[...]
        qn_r[...] = jnp.dot(local, wqr[...], preferred_element_type=jnp.float32) + bqr[...]

    grid = (pl.cdiv(N, NBK),)
    return pl.pallas_call(
        body,
        grid=grid,
        in_specs=[_rows(NBK, SIZE),
                  _full((SIZE, SIZE)), _full((1, SIZE)),
                  _full((SIZE, SIZE)), _full((1, SIZE)),
                  _full((SIZE, 2 * HA)), _full((1, 2 * HA)),
                  _full((SIZE, SIZE)), _full((1, SIZE))],
        out_specs=[_rows(NBK, SIZE), _rows(NBK, 2 * HA), _rows(NBK, SIZE)],
        out_shape=[jax.ShapeDtypeStruct((N, SIZE), jnp.float32),
                   jax.ShapeDtypeStruct((N, 2 * HA), jnp.float32),
                   jax.ShapeDtypeStruct((N, SIZE), jnp.float32)],
    )(feat, w0, b0, w1, b1, wkv, bkv, wq, bq)


def _node_prep_dec(feat, enc, seqtab, w0, b0, w1, b1, wkv, bkv, wkvs, wq, bq):
    def body(f_r, e_r, sq_r, w0r, b0r, w1r, b1r, wkvr, bkvr, wkvsr, wqr, bqr,
             inp_r, kva_r, kvb_r, qn_r):
        inp = jnp.maximum(f_r[...], 0.0)
        h = jnp.maximum(jnp.dot(inp, w0r[...], preferred_element_type=jnp.float32) + b0r[...], 0.0)
        local = jnp.maximum(jnp.dot(h, w1r[...], preferred_element_type=jnp.float32) + b1r[...], 0.0)
        inp_r[...] = inp
        kva_r[...] = (jnp.dot(local, wkvr[...], preferred_element_type=jnp.float32)
                      + jnp.dot(sq_r[...], wkvsr[...], preferred_element_type=jnp.float32)
                      + bkvr[...])
        kvb_r[...] = jnp.dot(e_r[...], wkvr[...], preferred_element_type=jnp.float32) + bkvr[...]
        qn_r[...] = jnp.dot(local, wqr[...], preferred_element_type=jnp.float32) + bqr[...]

    grid = (pl.cdiv(N, NBK),)
    return pl.pallas_call(
        body,
        grid=grid,
        in_specs=[_rows(NBK, SIZE), _rows(NBK, SIZE), _rows(NBK, SEQ_SIZE),
                  _full((SIZE, SIZE)), _full((1, SIZE)),
                  _full((SIZE, SIZE)), _full((1, SIZE)),
                  _full((SIZE, 2 * HA)), _full((1, 2 * HA)),
                  _full((SEQ_SIZE, 2 * HA)),
                  _full((SIZE, SIZE)), _full((1, SIZE))],
        out_specs=[_rows(NBK, SIZE), _rows(NBK, 2 * HA), _rows(NBK, 2 * HA),
                   _rows(NBK, SIZE)],
        out_shape=[jax.ShapeDtypeStruct((N, SIZE), jnp.float32),
                   jax.ShapeDtypeStruct((N, 2 * HA), jnp.float32),
                   jax.ShapeDtypeStruct((N, 2 * HA), jnp.float32),
                   jax.ShapeDtypeStruct((N, SIZE), jnp.float32)],
    )(feat, enc, seqtab, w0, b0, w1, b1, wkv, bkv, wkvs, wq, bq)


def _node_post(part, inp, wo, bo, wpost=None, bpost=None):
    final = wpost is not None

    def body(*args):
        if final:
            p_r, inp_r, wo_r, bo_r, wp_r, bp_r, out_r = args
        else:
            p_r, inp_r, wo_r, bo_r, out_r = args
        p = p_r[...]
        s = p[0] + p[1]
        ii = lax.broadcasted_iota(jnp.int32, (ACCW, HA), 0)
        jj = lax.broadcasted_iota(jnp.int32, (ACCW, HA), 1)
        pick_num = jnp.where((ii == jj) & (ii < HA), 1.0, 0.0)
        pick_den = jnp.where((ii >= HA) & (ii < HA + HEADS) & ((ii - HA) == jj // ATT), 1.0, 0.0)
        num = jnp.dot(s, pick_num, preferred_element_type=jnp.float32)
        den = jnp.dot(s, pick_den, preferred_element_type=jnp.float32)
        att_in = num / (den + 1e-30)
        att = jnp.dot(att_in, wo_r[...], preferred_element_type=jnp.float32) + bo_r[...]
        feat = inp_r[...] + att
        if final:
            out_r[...] = jnp.dot(feat, wp_r[...], preferred_element_type=jnp.float32) + bp_r[...]
        else:
            out_r[...] = feat

    grid = (pl.cdiv(N, NBK),)
    in_specs = [pl.BlockSpec((NC, NBK, ACCW), lambda i: (0, i, 0)),
                _rows(NBK, SIZE), _full((HA, SIZE)), _full((1, SIZE))]
    args = [part, inp, wo, bo]
    if final:
        in_specs += [_full((SIZE, OUT_SIZE)), _full((1, OUT_SIZE))]
        args += [wpost, bpost]
        out_w = OUT_SIZE
    else:
        out_w = SIZE
    return pl.pallas_call(
        body,
        grid=grid,
        in_specs=in_specs,
        out_specs=_rows(NBK, out_w),
        out_shape=jax.ShapeDtypeStruct((N, out_w), jnp.float32),
    )(*args)


def _row(b):
    return b.reshape(1, -1)


def kernel(features, distances, params, sequence, connections, indices):
    conn = connections.astype(jnp.int32)
    ind = indices.astype(jnp.int32)
    pad_rows = ((0, NCHP - NCH), (0, 0))
    conn2 = jnp.pad(conn.reshape(NCH, CHUNK), pad_rows)
    ind2 = jnp.pad(ind.reshape(NCH, CHUNK), pad_rows)

    distp = jnp.pad(distances, ((0, 0), (0, SIZE - 13)))
    src3, tgt3 = _sc_gather2(distp, conn2, distp, ind2)
    rel = _rel_tc(src3.reshape(E, SIZE), tgt3.reshape(E, SIZE))

    onehot = jax.nn.one_hot(sequence, OUT_SIZE, dtype=jnp.float32)
    feat, seqtab = _pre_tc(features, onehot,
                           params["pre"]["W"], _row(params["pre"]["b"]),
                           params["seq_emb"]["W"], _row(params["seq_emb"]["b"]))

    zeros_acc = jnp.zeros((N, ACCW), jnp.float32)

    def block_weights(bp, msg_extra):
        wk, wv = bp["k"]["W"], bp["v"]["W"]
        wkv = jnp.concatenate([wk[:SIZE], wv[:SIZE]], axis=1)
        bkv = jnp.concatenate([bp["k"]["b"], bp["v"]["b"]])
        wkr = jnp.pad(wk[SIZE:SIZE + 30], ((0, 2), (0, 0)))
        wvr = jnp.pad(wv[SIZE:SIZE + 30], ((0, 2), (0, 0)))
        wq = jnp.pad(bp["q"]["W"], ((0, 0), (0, SIZE - HA)))
        bq = jnp.pad(bp["q"]["b"], (0, SIZE - HA))
        extra = None
        if msg_extra:
            extra = jnp.concatenate([wk[SIZE + 30:], wv[SIZE + 30:]], axis=1)
        return wkv, bkv, wkr, wvr, wq, bq, extra

    for bp in params["enc_blocks"]:
        wkv, bkv, wkr, wvr, wq, bq, _ = block_weights(bp, False)
        inp, kvt, qn = _node_prep_enc(
            feat, bp["local"][0]["W"], _row(bp["local"][0]["b"]),
            bp["local"][1]["W"], _row(bp["local"][1]["b"]),
            wkv, _row(bkv), wq, _row(bq))
        kv3, q3 = _sc_gather2(kvt, conn2, qn, ind2)
        wvx = _edge_tc(kv3.reshape(E, 2 * HA), q3.reshape(E, SIZE), rel, wkr, wvr)
        part = _sc_scatter(wvx.reshape(NCH, CHUNK, ACCW), ind2, zeros_acc)
        feat = _node_post(part, inp, bp["o"]["W"], _row(bp["o"]["b"]))

    encoding = feat

    pre_mask = conn < jnp.arange(E, dtype=jnp.int32)
    idx_dec2 = jnp.pad(jnp.where(pre_mask, conn, conn + N).reshape(NCH, CHUNK), pad_rows)

    for bi, bp in enumerate(params["dec_blocks"]):
        wkv, bkv, wkr, wvr, wq, bq, wkvs = block_weights(bp, True)
        inp, kva, kvb, qn = _node_prep_dec(
            feat, encoding, seqtab,
            bp["local"][0]["W"], _row(bp["local"][0]["b"]),
            bp["local"][1]["W"], _row(bp["local"][1]["b"]),
            wkv, _row(bkv), wkvs, wq, _row(bq))
        kvt = jnp.concatenate([kva, kvb], axis=0)
        kv3, q3 = _sc_gather2(kvt, idx_dec2, qn, ind2)
        wvx = _edge_tc(kv3.reshape(E, 2 * HA), q3.reshape(E, SIZE), rel, wkr, wvr)
        part = _sc_scatter(wvx.reshape(NCH, CHUNK, ACCW), ind2, zeros_acc)
        if bi == len(params["dec_blocks"]) - 1:
            feat = _node_post(part, inp, bp["o"]["W"], _row(bp["o"]["b"]),
                              params["post"]["W"], _row(params["post"]["b"]))
        else:
            feat = _node_post(part, inp, bp["o"]["W"], _row(bp["o"]["b"]))

    return feat


# speedup vs baseline: 1.4082x; 1.1009x over previous
import functools

import jax
import jax.numpy as jnp
from jax import lax
from jax.experimental import pallas as pl
from jax.experimental.pallas import tpu as pltpu
from jax.experimental.pallas import tpu_sc as plsc

N = 10000
E = 320000
IN_SIZE = 128
OUT_SIZE = 20
SIZE = 128
SEQ_SIZE = 32
ATT = 16
HEADS = 4
HA = HEADS * ATT
KERNELS = 16
MAX_DIST = 20.0

NC = 2
NS = 16
NW = NC * NS
CHUNK = 128
MC = 2
NCH = E // CHUNK
RPW = 80
NCHP = NW * RPW
MCS = 1
NTILE = N // NS
ACCW = 128

NCHH = NCH // 2
RPWH = 40
NCHPH = NW * RPWH
EH = NCHH * CHUNK

EB = 2000
NBK = 512


def _sc_gather2(tab1, idx1p, tab2, idx2p, nch, rpw):
    d1 = tab1.shape[1]
    d2 = tab2.shape[1]
    mesh = plsc.VectorSubcoreMesh(core_axis_name="c", subcore_axis_name="s", num_cores=NC, num_subcores=NS)
    out_type = (
        jax.ShapeDtypeStruct((nch, CHUNK, d1), tab1.dtype),
        jax.ShapeDtypeStruct((nch, CHUNK, d2), tab2.dtype),
    )
    scratch = [
        pltpu.VMEM((rpw, CHUNK), jnp.int32),
        pltpu.VMEM((rpw, CHUNK), jnp.int32),
        pltpu.VMEM((MC, CHUNK, d1), tab1.dtype),
        pltpu.VMEM((MC, CHUNK, d2), tab2.dtype),
        pltpu.SemaphoreType.DMA,
        pltpu.SemaphoreType.DMA,
    ]

    def body(t1_h, i1_h, t2_h, i2_h, o1_h, o2_h, i1_v, i2_v, b1_v, b2_v, s1, s2):
        c = lax.axis_index("c")
        s = lax.axis_index("s")
        wid = s * NC + c
        row0 = wid * rpw
        pltpu.sync_copy(i1_h.at[pl.ds(row0, rpw)], i1_v)
        pltpu.sync_copy(i2_h.at[pl.ds(row0, rpw)], i2_v)
        nj = jnp.minimum(rpw // MC, jnp.maximum(0, (nch - row0) // MC))

        def jbody(j, carry):
            base = j * MC
            gbase = row0 + base
            waits = []
            for i in range(MC):
                waits.append(pltpu.async_copy(t1_h.at[i1_v.at[base + i]], b1_v.at[i], s1))
                waits.append(pltpu.async_copy(t2_h.at[i2_v.at[base + i]], b2_v.at[i], s2))
            for w in waits:
                w.wait()
            pltpu.sync_copy(b1_v, o1_h.at[pl.ds(gbase, MC)])
            pltpu.sync_copy(b2_v, o2_h.at[pl.ds(gbase, MC)])
            return carry

        lax.fori_loop(0, nj, jbody, 0)

    f = pl.kernel(body, out_type=out_type, mesh=mesh, scratch_types=scratch,
                  compiler_params=pltpu.CompilerParams(use_tc_tiling_on_sc=False))
    return f(tab1, idx1p, tab2, idx2p)


def _sc_scatter(wv3, ind2, zeros_h, nch, rpw):
    mesh = plsc.VectorSubcoreMesh(core_axis_name="c", subcore_axis_name="s", num_cores=NC, num_subcores=NS)
    out_type = jax.ShapeDtypeStruct((NC, N, ACCW), jnp.float32)
    scratch = [
        pltpu.VMEM_SHARED((N, ACCW), jnp.float32),
        pltpu.VMEM((rpw, CHUNK), jnp.int32),
        pltpu.VMEM((2, MCS, CHUNK, ACCW), jnp.float32),
        pltpu.SemaphoreType.DMA((2,)),
    ]

    def body(wv_h, ind_h, z_h, part_h, acc, ind_v, wv_v, sem):
        c = lax.axis_index("c")
        s = lax.axis_index("s")
        wid = s * NC + c
        r0 = s * NTILE
        row0 = wid * rpw
        pltpu.sync_copy(z_h.at[pl.ds(r0, NTILE)], acc.at[pl.ds(r0, NTILE)])
        pltpu.sync_copy(ind_h.at[pl.ds(row0, rpw)], ind_v)
        plsc.subcore_barrier()
        nj = jnp.minimum(rpw // MCS, jnp.maximum(0, (nch - row0) // MCS))

        def adds(base, slot):
            for i in range(MCS):
                pltpu.sync_copy(wv_v.at[slot, i], acc.at[ind_v.at[base + i]], add=True)

        def jbody(jj, carry):
            j0 = 2 * jj
            ha = pltpu.async_copy(wv_h.at[pl.ds(row0 + j0 * MCS, MCS)], wv_v.at[0], sem.at[0])

            @pl.when(j0 + 1 < nj)
            def _():
                pltpu.async_copy(wv_h.at[pl.ds(row0 + (j0 + 1) * MCS, MCS)], wv_v.at[1], sem.at[1])

            ha.wait()
            adds(j0 * MCS, 0)

            @pl.when(j0 + 1 < nj)
            def _():
                pltpu.make_async_copy(wv_h.at[pl.ds(0, MCS)], wv_v.at[1], sem.at[1]).wait()
                adds((j0 + 1) * MCS, 1)

            return carry

        lax.fori_loop(0, (nj + 1) // 2, jbody, 0)
        plsc.subcore_barrier()
        pltpu.sync_copy(acc.at[pl.ds(r0, NTILE)], part_h.at[c, pl.ds(r0, NTILE)])

    f = pl.kernel(body, out_type=out_type, mesh=mesh, scratch_types=scratch,
                  compiler_params=pltpu.CompilerParams(use_tc_tiling_on_sc=False))
    return f(wv3, ind2, zeros_h)


def _full(shape):
    return pl.BlockSpec(shape, lambda i: tuple(0 for _ in shape))


def _rows(bs, width):
    return pl.BlockSpec((bs, width), lambda i: (i, 0))


def _rel_tc(src_d, tgt_d):
    def body(src_r, tgt_r, out_r):
        src = src_r[...]
        tgt = tgt_r[...]
        diff = tgt[:, 0:3] - src[:, 0:3]
        d2 = jnp.sum(diff * diff, axis=1, keepdims=True)
        dist = jnp.sqrt(d2 + 1e-12)
        direction = diff / (dist + 1e-6)
        centers = jnp.arange(KERNELS, dtype=jnp.int32).astype(jnp.float32) * (MAX_DIST / (KERNELS - 1))
        sigma = MAX_DIST / KERNELS
        rbf = jnp.exp(-((dist - centers[None, :]) ** 2) / (2.0 * sigma * sigma))
        rots = []
        for i in range(3):
            xo = tgt[:, 3 + 3 * i:6 + 3 * i]
            for j in range(3):
                yo = src[:, 3 + 3 * j:6 + 3 * j]
                rots.append(jnp.sum(xo * yo, axis=1, keepdims=True))
        di = (tgt[:, 12:13] - src[:, 12:13]) * 0.1
        ds = jnp.sin(di)
        dc = jnp.cos(di)
        pad = jnp.zeros((src.shape[0], 2), jnp.float32)
        out_r[...] = jnp.concatenate([rbf, direction] + rots + [ds, dc, pad], axis=1)

    rows = src_d.shape[0]
    return pl.pallas_call(
        body,
        grid=(rows // EB,),
        in_specs=[_rows(EB, 128), _rows(EB, 128)],
        out_specs=_rows(EB, 32),
        out_shape=jax.ShapeDtypeStruct((rows, 32), jnp.float32),
    )(src_d, tgt_d)


def _edge_tc(kv_g, q_g, rel, wkr, wvr):
    def body(kv_r, q_r, rel_r, wkr_r, wvr_r, out_r):
        kv = kv_r[...]
        q = q_r[...][:, :HA]
        relv = rel_r[...]
        k = kv[:, :HA] + jnp.dot(relv, wkr_r[...], preferred_element_type=jnp.float32)
        v = kv[:, HA:] + jnp.dot(relv, wvr_r[...], preferred_element_type=jnp.float32)
        t = q * k
        dd = lax.broadcasted_iota(jnp.int32, (HA, HEADS), 0) // ATT
        hh = lax.broadcasted_iota(jnp.int32, (HA, HEADS), 1)
        collapse = (dd == hh).astype(jnp.float32)
        logits = jnp.dot(t, collapse, preferred_element_type=jnp.float32) * 0.25
        w = jnp.exp(logits)
        wb = jnp.dot(w, collapse.T, preferred_element_type=jnp.float32)
        wv = wb * v
        pad = jnp.zeros((kv.shape[0], ACCW - HA - HEADS), jnp.float32)
        out_r[...] = jnp.concatenate([wv, w, pad], axis=1)

    rows = kv_g.shape[0]
    return pl.pallas_call(
        body,
        grid=(rows // EB,),
        in_specs=[_rows(EB, 128), _rows(EB, 128), _rows(EB, 32),
                  _full((32, HA)), _full((32, HA))],
        out_specs=_rows(EB, ACCW),
        out_shape=jax.ShapeDtypeStruct((rows, ACCW), jnp.float32),
    )(kv_g, q_g, rel, wkr, wvr)


def _pre_tc(features, onehot, wpre, bpre, wse, bse):
    def body(f_r, oh_r, wp_r, bp_r, ws_r, bs_r, out_r, seq_r):
        out_r[...] = jnp.dot(f_r[...], wp_r[...],
                             preferred_element_type=jnp.float32) + bp_r[...]
        seq_r[...] = jnp.dot(oh_r[...], ws_r[...],
                             preferred_element_type=jnp.float32) + bs_r[...]

    grid = (pl.cdiv(N, NBK),)
    return pl.pallas_call(
        body,
        grid=grid,
        in_specs=[_rows(NBK, IN_SIZE), _rows(NBK, OUT_SIZE),
                  _full((IN_SIZE, SIZE)), _full((1, SIZE)),
                  _full((OUT_SIZE, SEQ_SIZE)), _full((1, SEQ_SIZE))],
        out_specs=[_rows(NBK, SIZE), _rows(NBK, SEQ_SIZE)],
        out_shape=[jax.ShapeDtypeStruct((N, SIZE), jnp.float32),
                   jax.ShapeDtypeStruct((N, SEQ_SIZE), jnp.float32)],
    )(features, onehot, wpre, bpre, wse, bse)


def _node_prep_enc(feat, w0, b0, w1, b1, wkv, bkv, wq, bq):
    def body(f_r, w0r, b0r, w1r, b1r, wkvr, bkvr, wqr, bqr, inp_r, kv_r, qn_r):
        inp = jnp.maximum(f_r[...], 0.0)
        h = jnp.maximum(jnp.dot(inp, w0r[...], preferred_element_type=jnp.float32) + b0r[...], 0.0)
        local = jnp.maximum(jnp.dot(h, w1r[...], preferred_element_type=jnp.float32) + b1r[...], 0.0)
        inp_r[...] = inp
        kv_r[...] = jnp.dot(local, wkvr[...], preferred_element_type=jnp.float32) + bkvr[...]
        qn_r[...] = jnp.dot(local, wqr[...], preferred_element_type=jnp.float32) + bqr[...]

    grid = (pl.cdiv(N, NBK),)
    return pl.pallas_call(
        body,
        grid=grid,
        in_specs=[_rows(NBK, SIZE),
                  _full((SIZE, SIZE)), _full((1, SIZE)),
                  _full((SIZE, SIZE)), _full((1, SIZE)),
                  _full((SIZE, 2 * HA)), _full((1, 2 * HA)),
                  _full((SIZE, SIZE)), _full((1, SIZE))],
        out_specs=[_rows(NBK, SIZE), _rows(NBK, 2 * HA), _rows(NBK, SIZE)],
        out_shape=[jax.ShapeDtypeStruct((N, SIZE), jnp.float32),
                   jax.ShapeDtypeStruct((N, 2 * HA), jnp.float32),
                   jax.ShapeDtypeStruct((N, SIZE), jnp.float32)],
    )(feat, w0, b0, w1, b1, wkv, bkv, wq, bq)


def _node_prep_dec(feat, enc, seqtab, w0, b0, w1, b1, wkv, bkv, wkvs, wq, bq):
    def body(f_r, e_r, sq_r, w0r, b0r, w1r, b1r, wkvr, bkvr, wkvsr, wqr, bqr,
             inp_r, kva_r, kvb_r, qn_r):
        inp = jnp.maximum(f_r[...], 0.0)
        h = jnp.maximum(jnp.dot(inp, w0r[...], preferred_element_type=jnp.float32) + b0r[...], 0.0)
        local = jnp.maximum(jnp.dot(h, w1r[...], preferred_element_type=jnp.float32) + b1r[...], 0.0)
        inp_r[...] = inp
        kva_r[...] = (jnp.dot(local, wkvr[...], preferred_element_type=jnp.float32)
                      + jnp.dot(sq_r[...], wkvsr[...], preferred_element_type=jnp.float32)
                      + bkvr[...])
        kvb_r[...] = jnp.dot(e_r[...], wkvr[...], preferred_element_type=jnp.float32) + bkvr[...]
        qn_r[...] = jnp.dot(local, wqr[...], preferred_element_type=jnp.float32) + bqr[...]

    grid = (pl.cdiv(N, NBK),)
    return pl.pallas_call(
        body,
        grid=grid,
        in_specs=[_rows(NBK, SIZE), _rows(NBK, SIZE), _rows(NBK, SEQ_SIZE),
                  _full((SIZE, SIZE)), _full((1, SIZE)),
                  _full((SIZE, SIZE)), _full((1, SIZE)),
                  _full((SIZE, 2 * HA)), _full((1, 2 * HA)),
                  _full((SEQ_SIZE, 2 * HA)),
                  _full((SIZE, SIZE)), _full((1, SIZE))],
        out_specs=[_rows(NBK, SIZE), _rows(NBK, 2 * HA), _rows(NBK, 2 * HA),
                   _rows(NBK, SIZE)],
        out_shape=[jax.ShapeDtypeStruct((N, SIZE), jnp.float32),
                   jax.ShapeDtypeStruct((N, 2 * HA), jnp.float32),
                   jax.ShapeDtypeStruct((N, 2 * HA), jnp.float32),
                   jax.ShapeDtypeStruct((N, SIZE), jnp.float32)],
    )(feat, enc, seqtab, w0, b0, w1, b1, wkv, bkv, wkvs, wq, bq)


def _node_post(part0, part1, inp, wo, bo, wpost=None, bpost=None):
    final = wpost is not None

    def body(*args):
        if final:
            p0_r, p1_r, inp_r, wo_r, bo_r, wp_r, bp_r, out_r = args
        else:
            p0_r, p1_r, inp_r, wo_r, bo_r, out_r = args
        p0 = p0_r[...]
        p1 = p1_r[...]
        s = (p0[0] + p0[1]) + (p1[0] + p1[1])
        ii = lax.broadcasted_iota(jnp.int32, (ACCW, HA), 0)
        jj = lax.broadcasted_iota(jnp.int32, (ACCW, HA), 1)
        pick_num = jnp.where((ii == jj) & (ii < HA), 1.0, 0.0)
        pick_den = jnp.where((ii >= HA) & (ii < HA + HEADS) & ((ii - HA) == jj // ATT), 1.0, 0.0)
        num = jnp.dot(s, pick_num, preferred_element_type=jnp.float32)
        den = jnp.dot(s, pick_den, preferred_element_type=jnp.float32)
        att_in = num / (den + 1e-30)
        att = jnp.dot(att_in, wo_r[...], preferred_element_type=jnp.float32) + bo_r[...]
        feat = inp_r[...] + att
        if final:
            out_r[...] = jnp.dot(feat, wp_r[...], preferred_element_type=jnp.float32) + bp_r[...]
        else:
            out_r[...] = feat

    grid = (pl.cdiv(N, NBK),)
    in_specs = [pl.BlockSpec((NC, NBK, ACCW), lambda i: (0, i, 0)),
                pl.BlockSpec((NC, NBK, ACCW), lambda i: (0, i, 0)),
                _rows(NBK, SIZE), _full((HA, SIZE)), _full((1, SIZE))]
    args = [part0, part1, inp, wo, bo]
    if final:
        in_specs += [_full((SIZE, OUT_SIZE)), _full((1, OUT_SIZE))]
        args += [wpost, bpost]
        out_w = OUT_SIZE
    else:
        out_w = SIZE
    return pl.pallas_call(
        body,
        grid=grid,
        in_specs=in_specs,
        out_specs=_rows(NBK, out_w),
        out_shape=jax.ShapeDtypeStruct((N, out_w), jnp.float32),
    )(*args)


def _row(b):
    return b.reshape(1, -1)


def _edge_halves(kvt, kvidx2, qn, ind2, rel, wkr, wvr, zeros_acc):
    kv0, q0 = _sc_gather2(kvt, kvidx2[0], qn, ind2[0], NCHH, RPWH)
    wvx0 = _edge_tc(kv0.reshape(EH, 2 * HA), q0.reshape(EH, SIZE), rel[0], wkr, wvr)
    kv1, q1 = _sc_gather2(kvt, kvidx2[1], qn, ind2[1], NCHH, RPWH)
    part0 = _sc_scatter(wvx0.reshape(NCHH, CHUNK, ACCW), ind2[0], zeros_acc, NCHH, RPWH)
    wvx1 = _edge_tc(kv1.reshape(EH, 2 * HA), q1.reshape(EH, SIZE), rel[1], wkr, wvr)
    part1 = _sc_scatter(wvx1.reshape(NCHH, CHUNK, ACCW), ind2[1], zeros_acc, NCHH, RPWH)
    return part0, part1


def kernel(features, distances, params, sequence, connections, indices):
    conn = connections.astype(jnp.int32)
    ind = indices.astype(jnp.int32)
    c2 = conn.reshape(NCH, CHUNK)
    i2 = ind.reshape(NCH, CHUNK)
    pad_h = ((0, NCHPH - NCHH), (0, 0))
    conn2 = [jnp.pad(c2[:NCHH], pad_h), jnp.pad(c2[NCHH:], pad_h)]
    ind2 = [jnp.pad(i2[:NCHH], pad_h), jnp.pad(i2[NCHH:], pad_h)]

    distp = jnp.pad(distances, ((0, 0), (0, SIZE - 13)))
    rel = []
    for h in range(2):
        s3, t3 = _sc_gather2(distp, conn2[h], distp, ind2[h], NCHH, RPWH)
        rel.append(_rel_tc(s3.reshape(EH, SIZE), t3.reshape(EH, SIZE)))

    onehot = jax.nn.one_hot(sequence, OUT_SIZE, dtype=jnp.float32)
    feat, seqtab = _pre_tc(features, onehot,
                           params["pre"]["W"], _row(params["pre"]["b"]),
                           params["seq_emb"]["W"], _row(params["seq_emb"]["b"]))

    zeros_acc = jnp.zeros((N, ACCW), jnp.float32)

    def block_weights(bp, msg_extra):
        wk, wv = bp["k"]["W"], bp["v"]["W"]
        wkv = jnp.concatenate([wk[:SIZE], wv[:SIZE]], axis=1)
        bkv = jnp.concatenate([bp["k"]["b"], bp["v"]["b"]])
        wkr = jnp.pad(wk[SIZE:SIZE + 30], ((0, 2), (0, 0)))
        wvr = jnp.pad(wv[SIZE:SIZE + 30], ((0, 2), (0, 0)))
        wq = jnp.pad(bp["q"]["W"], ((0, 0), (0, SIZE - HA)))
        bq = jnp.pad(bp["q"]["b"], (0, SIZE - HA))
        extra = None
        if msg_extra:
            extra = jnp.concatenate([wk[SIZE + 30:], wv[SIZE + 30:]], axis=1)
        return wkv, bkv, wkr, wvr, wq, bq, extra

    for bp in params["enc_blocks"]:
        wkv, bkv, wkr, wvr, wq, bq, _ = block_weights(bp, False)
        inp, kvt, qn = _node_prep_enc(
            feat, bp["local"][0]["W"], _row(bp["local"][0]["b"]),
            bp["local"][1]["W"], _row(bp["local"][1]["b"]),
            wkv, _row(bkv), wq, _row(bq))
        parts = _edge_halves(kvt, conn2, qn, ind2, rel, wkr, wvr, zeros_acc)
        feat = _node_post(parts[0], parts[1], inp, bp["o"]["W"], _row(bp["o"]["b"]))

    encoding = feat

    pre_mask = conn < jnp.arange(E, dtype=jnp.int32)
    d2 = jnp.where(pre_mask, conn, conn + N).reshape(NCH, CHUNK)
    idx_dec2 = [jnp.pad(d2[:NCHH], pad_h), jnp.pad(d2[NCHH:], pad_h)]

    for bi, bp in enumerate(params["dec_blocks"]):
        wkv, bkv, wkr, wvr, wq, bq, wkvs = block_weights(bp, True)
        inp, kva, kvb, qn = _node_prep_dec(
            feat, encoding, seqtab,
            bp["local"][0]["W"], _row(bp["local"][0]["b"]),
            bp["local"][1]["W"], _row(bp["local"][1]["b"]),
            wkv, _row(bkv), wkvs, wq, _row(bq))
        kvt = jnp.concatenate([kva, kvb], axis=0)
        parts = _edge_halves(kvt, idx_dec2, qn, ind2, rel, wkr, wvr, zeros_acc)
        if bi == len(params["dec_blocks"]) - 1:
            feat = _node_post(parts[0], parts[1], inp, bp["o"]["W"], _row(bp["o"]["b"]),
                              params["post"]["W"], _row(params["post"]["b"]))
        else:
            feat = _node_post(parts[0], parts[1], inp, bp["o"]["W"], _row(bp["o"]["b"]))

    return feat
